# baseline (speedup 1.0000x reference)
import functools

import jax
import jax.numpy as jnp
from jax import lax
from jax.experimental import pallas as pl
from jax.experimental.pallas import tpu as pltpu
from jax.experimental.pallas import tpu_sc as plsc

_NC = 2
_NS = 16
_LANE = 16
_ROW = 512
_GRP = 1


@functools.lru_cache(maxsize=None)
def _make_segsum(n: int, ep: int):
    rows_per_tile = ep // _ROW // _NS
    groups = rows_per_tile // _GRP
    npt = -(-n // (_NS * 128)) * 128
    zr = 128
    zcopies = npt // zr
    mesh = plsc.VectorSubcoreMesh(core_axis_name="c", subcore_axis_name="s")

    @functools.partial(
        pl.kernel,
        out_type=jax.ShapeDtypeStruct((n, _NC, _LANE), jnp.float32),
        mesh=mesh,
        scratch_types=[
            pltpu.VMEM_SHARED((n + 8, _LANE), jnp.float32),
            pltpu.VMEM((4, 2, _ROW), jnp.int32),
            pltpu.VMEM((2, _ROW, _LANE), jnp.float32),
            pltpu.VMEM((zr, _LANE), jnp.float32),
            pltpu.SemaphoreType.DMA((4,)),
            pltpu.SemaphoreType.DMA((2,)),
            pltpu.SemaphoreType.DMA((2,)),
        ],
        compiler_params=pltpu.CompilerParams(use_tc_tiling_on_sc=False),
    )
    def segsum(hp2, comb0r, comb1r, out, acc, cbuf, rows, zbuf,
               csem, gsem, ssem):
        c = lax.axis_index("c")
        s = lax.axis_index("s")
        start = pl.multiple_of(jnp.minimum(s * npt, n - npt), 8)

        def zero_body(i, carry):
            zbuf[i] = jnp.zeros((_LANE,), jnp.float32)
            return carry

        lax.fori_loop(0, zr, zero_body, 0)
        for k in range(zcopies):
            pltpu.sync_copy(zbuf, acc.at[pl.ds(start + k * zr, zr)])
        plsc.subcore_barrier()

        def fire_idx(g, p4):
            row0 = s * rows_per_tile + g

            @pl.when(c == 0)
            def _():
                pltpu.async_copy(comb0r.at[row0], cbuf.at[p4], csem.at[p4])

            @pl.when(c == 1)
            def _():
                pltpu.async_copy(comb1r.at[row0], cbuf.at[p4], csem.at[p4])

        def wait_idx(p4):
            pltpu.make_async_copy(comb0r.at[0], cbuf.at[p4], csem.at[p4]).wait()

        def fire_gather(p2, p4):
            pltpu.async_copy(hp2.at[cbuf.at[p4, 0]], rows.at[p2], gsem.at[p2])

        def drain_gather(p2):
            pltpu.make_async_copy(hp2.at[pl.ds(0, _ROW)], rows.at[p2],
                                  gsem.at[p2]).wait()

        def fire_scatter(p2, p4):
            pltpu.async_copy(rows.at[p2], acc.at[cbuf.at[p4, 1]], ssem.at[p2],
                             add=True)

        def drain_scatter(p2):
            pltpu.make_async_copy(rows.at[p2], acc.at[pl.ds(0, _ROW)],
                                  ssem.at[p2]).wait()

        fire_idx(0, 0)
        wait_idx(0)
        fire_gather(0, 0)
        fire_idx(1, 1)

        def group_body(g, carry):
            p2 = lax.rem(g, 2)
            q2 = 1 - p2
            p4 = lax.rem(g, 4)
            pm4 = lax.rem(g + 3, 4)
            pn4 = lax.rem(g + 1, 4)

            @pl.when(g >= 2)
            def _():
                drain_scatter(p2)

            drain_gather(q2)
            fire_scatter(q2, pm4)
            @pl.when(g + 1 < groups)
            def _():
                fire_idx(g + 1, pn4)

            wait_idx(p4)
            fire_gather(p2, p4)
            return carry

        lax.fori_loop(1, groups, group_body, 0)
        last2 = (groups - 1) % 2
        last4 = (groups - 1) % 4
        drain_gather(last2)
        fire_scatter(last2, last4)
        drain_scatter(1 - last2)
        drain_scatter(last2)
        plsc.subcore_barrier()
        pltpu.sync_copy(acc.at[pl.ds(start, npt)],
                        out.at[pl.ds(start, npt), c])

    return segsum


def _mlp_block(h_ref, agg_ref, wd1_ref, b1d_ref, wd2_ref, b2d_ref):
    z = h_ref[...] + agg_ref[...]
    t = jnp.dot(z, wd1_ref[...], preferred_element_type=jnp.float32) + b1d_ref[...]
    t = jnp.maximum(t, 0.0)
    u = jnp.dot(t, wd2_ref[...], preferred_element_type=jnp.float32) + b2d_ref[...]
    return jnp.maximum(u, 0.0)


def _mlp_body(h_ref, agg_ref, wd1_ref, b1d_ref, wd2_ref, b2d_ref, out_ref):
    out_ref[...] = _mlp_block(h_ref, agg_ref, wd1_ref, b1d_ref, wd2_ref, b2d_ref)


def _mlp_final_body(h_ref, agg_ref, wd1_ref, b1d_ref, wd2_ref, b2d_ref,
                    wfd_ref, bfd_ref, y_ref):
    u = _mlp_block(h_ref, agg_ref, wd1_ref, b1d_ref, wd2_ref, b2d_ref)
    y_ref[...] = jnp.dot(u, wfd_ref[...], preferred_element_type=jnp.float32) + bfd_ref[...]


@functools.lru_cache(maxsize=None)
def _make_mlp(n4: int, blk: int, final: bool):
    grid = (n4 // blk,)
    h_spec = pl.BlockSpec((blk, 128), lambda i: (i, 0))
    w_spec = pl.BlockSpec((128, 128), lambda i: (0, 0))
    b_spec = pl.BlockSpec((128,), lambda i: (0,))
    in_specs = [h_spec, h_spec, w_spec, b_spec, w_spec, b_spec]
    if final:
        in_specs += [pl.BlockSpec((128, 4), lambda i: (0, 0)),
                     pl.BlockSpec((4,), lambda i: (0,))]
        out_spec = pl.BlockSpec((blk, 4), lambda i: (i, 0))
        out_shape = jax.ShapeDtypeStruct((n4, 4), jnp.float32)
        body = _mlp_final_body
    else:
        out_spec = h_spec
        out_shape = jax.ShapeDtypeStruct((n4, 128), jnp.float32)
        body = _mlp_body
    return pl.pallas_call(
        body, grid=grid, in_specs=in_specs, out_specs=out_spec, out_shape=out_shape
    )


def kernel(x, edge_index, W1_0, b1_0, W2_0, b2_0, W1_1, b1_1, W2_1, b2_1,
           W1_2, b1_2, W2_2, b2_2, Wf, bf):
    n, f = x.shape
    e = edge_index.shape[1]
    assert f == 32 and n % 8 == 0

    tile_quant = _NS * _ROW * _GRP
    ep = -(-e // tile_quant) * tile_quant
    src = edge_index[0]
    dst = edge_index[1]
    if ep != e:
        src = jnp.concatenate([src, jnp.zeros((ep - e,), jnp.int32)])
        dst = jnp.concatenate([dst, jnp.full((ep - e,), n, jnp.int32)])
    src2 = (src * 2).reshape(-1, _ROW)
    dst = dst.reshape(-1, _ROW)
    comb0 = jnp.stack([src2, dst], axis=1)
    comb1 = jnp.stack([src2 + 1, dst], axis=1)

    eye4 = jnp.eye(4, dtype=jnp.float32)
    params = []
    for w1, b1, w2, b2 in [(W1_0, b1_0, W2_0, b2_0), (W1_1, b1_1, W2_1, b2_1),
                           (W1_2, b1_2, W2_2, b2_2)]:
        params.append((jnp.kron(eye4, w1), jnp.tile(b1, 4),
                       jnp.kron(eye4, w2), jnp.tile(b2, 4)))
    wfd = jnp.kron(eye4, Wf)
    bfd = jnp.tile(bf, 4)

    segsum = _make_segsum(n, ep)
    mlp = _make_mlp(n // 4, 1000, False)
    mlp_final = _make_mlp(n // 4, 1000, True)

    h4 = x.reshape(n // 4, 128)
    for wd1, b1d, wd2, b2d in params[:2]:
        agg = segsum(h4.reshape(2 * n, _LANE), comb0, comb1)
        h4 = mlp(h4, agg.reshape(n // 4, 128), wd1, b1d, wd2, b2d)
    wd1, b1d, wd2, b2d = params[2]
    agg = segsum(h4.reshape(2 * n, _LANE), comb0, comb1)
    y4 = mlp_final(h4, agg.reshape(n // 4, 128), wd1, b1d, wd2, b2d, wfd, bfd)
    return y4.reshape(n, 1)


# speedup vs baseline: 1.0276x; 1.0276x over previous
import functools

import jax
import jax.numpy as jnp
from jax import lax
from jax.experimental import pallas as pl
from jax.experimental.pallas import tpu as pltpu
from jax.experimental.pallas import tpu_sc as plsc

_NC = 2
_NS = 16
_LANE = 16
_ROW = 512
_GRP = 1


@functools.lru_cache(maxsize=None)
def _make_segsum(n: int, ep: int):
    rows_per_tile = ep // _ROW // _NS
    groups = rows_per_tile // _GRP
    npt = -(-n // (_NS * 128)) * 128
    zr = 128
    zcopies = npt // zr
    mesh = plsc.VectorSubcoreMesh(core_axis_name="c", subcore_axis_name="s")

    @functools.partial(
        pl.kernel,
        out_type=jax.ShapeDtypeStruct((n, _NC, _LANE), jnp.float32),
        mesh=mesh,
        scratch_types=[
            pltpu.VMEM_SHARED((n + 8, _LANE), jnp.float32),
            pltpu.VMEM((2, _ROW), jnp.int32),
            pltpu.VMEM((3, _ROW), jnp.int32),
            pltpu.VMEM((2, _ROW, _LANE), jnp.float32),
            pltpu.VMEM((zr, _LANE), jnp.float32),
            pltpu.SemaphoreType.DMA((2,)),
            pltpu.SemaphoreType.DMA((3,)),
            pltpu.SemaphoreType.DMA((2,)),
            pltpu.SemaphoreType.DMA((2,)),
        ],
        compiler_params=pltpu.CompilerParams(use_tc_tiling_on_sc=False),
    )
    def segsum(hp2, src0r, src1r, dstr, out, acc, srcb, dstb, rows, zbuf,
               issem, idsem, gsem, ssem):
        c = lax.axis_index("c")
        s = lax.axis_index("s")
        start = pl.multiple_of(jnp.minimum(s * npt, n - npt), 8)

        def zero_body(i, carry):
            zbuf[i] = jnp.zeros((_LANE,), jnp.float32)
            return carry

        lax.fori_loop(0, zr, zero_body, 0)
        for k in range(zcopies):
            pltpu.sync_copy(zbuf, acc.at[pl.ds(start + k * zr, zr)])
        plsc.subcore_barrier()

        def fire_idx(g, p2, p3):
            row0 = s * rows_per_tile + g

            @pl.when(c == 0)
            def _():
                pltpu.async_copy(src0r.at[row0], srcb.at[p2], issem.at[p2])

            @pl.when(c == 1)
            def _():
                pltpu.async_copy(src1r.at[row0], srcb.at[p2], issem.at[p2])

            pltpu.async_copy(dstr.at[row0], dstb.at[p3], idsem.at[p3])

        def wait_idx(p2, p3):
            pltpu.make_async_copy(src0r.at[0], srcb.at[p2], issem.at[p2]).wait()
            pltpu.make_async_copy(dstr.at[0], dstb.at[p3], idsem.at[p3]).wait()

        def fire_gather(p2):
            pltpu.async_copy(hp2.at[srcb.at[p2]], rows.at[p2], gsem.at[p2])

        def drain_gather(p2):
            pltpu.make_async_copy(hp2.at[srcb.at[p2]], rows.at[p2],
                                  gsem.at[p2]).wait()

        def fire_scatter(p2, p3):
            pltpu.async_copy(rows.at[p2], acc.at[dstb.at[p3]], ssem.at[p2],
                             add=True)

        def drain_scatter(p2):
            pltpu.make_async_copy(rows.at[p2], acc.at[pl.ds(0, _ROW)],
                                  ssem.at[p2]).wait()

        fire_idx(0, 0, 0)
        wait_idx(0, 0)
        fire_gather(0)
        fire_idx(1, 1, 1)

        def group_body(g, carry):
            p2 = lax.rem(g, 2)
            q2 = 1 - p2
            p3 = lax.rem(g, 3)
            pm3 = lax.rem(g + 2, 3)
            pn3 = lax.rem(g + 1, 3)

            @pl.when(g >= 2)
            def _():
                drain_scatter(p2)

            drain_gather(q2)
            fire_scatter(q2, pm3)
            @pl.when(g + 1 < groups)
            def _():
                fire_idx(g + 1, q2, pn3)

            wait_idx(p2, p3)
            fire_gather(p2)
            return carry

        lax.fori_loop(1, groups, group_body, 0)
        last2 = (groups - 1) % 2
        last3 = (groups - 1) % 3
        drain_gather(last2)
        fire_scatter(last2, last3)
        drain_scatter(1 - last2)
        drain_scatter(last2)
        plsc.subcore_barrier()
        pltpu.sync_copy(acc.at[pl.ds(start, npt)],
                        out.at[pl.ds(start, npt), c])

    return segsum


def _mlp_block(h_ref, agg_ref, wd1_ref, b1d_ref, wd2_ref, b2d_ref):
    z = h_ref[...] + agg_ref[...]
    t = jnp.dot(z, wd1_ref[...], preferred_element_type=jnp.float32) + b1d_ref[...]
    t = jnp.maximum(t, 0.0)
    u = jnp.dot(t, wd2_ref[...], preferred_element_type=jnp.float32) + b2d_ref[...]
    return jnp.maximum(u, 0.0)


def _mlp_body(h_ref, agg_ref, wd1_ref, b1d_ref, wd2_ref, b2d_ref, out_ref):
    out_ref[...] = _mlp_block(h_ref, agg_ref, wd1_ref, b1d_ref, wd2_ref, b2d_ref)


def _mlp_final_body(h_ref, agg_ref, wd1_ref, b1d_ref, wd2_ref, b2d_ref,
                    wfd_ref, bfd_ref, y_ref):
    u = _mlp_block(h_ref, agg_ref, wd1_ref, b1d_ref, wd2_ref, b2d_ref)
    y_ref[...] = jnp.dot(u, wfd_ref[...], preferred_element_type=jnp.float32) + bfd_ref[...]


@functools.lru_cache(maxsize=None)
def _make_mlp(n4: int, blk: int, final: bool):
    grid = (n4 // blk,)
    h_spec = pl.BlockSpec((blk, 128), lambda i: (i, 0))
    w_spec = pl.BlockSpec((128, 128), lambda i: (0, 0))
    b_spec = pl.BlockSpec((128,), lambda i: (0,))
    in_specs = [h_spec, h_spec, w_spec, b_spec, w_spec, b_spec]
    if final:
        in_specs += [pl.BlockSpec((128, 4), lambda i: (0, 0)),
                     pl.BlockSpec((4,), lambda i: (0,))]
        out_spec = pl.BlockSpec((blk, 4), lambda i: (i, 0))
        out_shape = jax.ShapeDtypeStruct((n4, 4), jnp.float32)
        body = _mlp_final_body
    else:
        out_spec = h_spec
        out_shape = jax.ShapeDtypeStruct((n4, 128), jnp.float32)
        body = _mlp_body
    return pl.pallas_call(
        body, grid=grid, in_specs=in_specs, out_specs=out_spec, out_shape=out_shape
    )


def kernel(x, edge_index, W1_0, b1_0, W2_0, b2_0, W1_1, b1_1, W2_1, b2_1,
           W1_2, b1_2, W2_2, b2_2, Wf, bf):
    n, f = x.shape
    e = edge_index.shape[1]
    assert f == 32 and n % 8 == 0

    tile_quant = _NS * _ROW * _GRP
    ep = -(-e // tile_quant) * tile_quant
    src = edge_index[0]
    dst = edge_index[1]
    if ep != e:
        src = jnp.concatenate([src, jnp.zeros((ep - e,), jnp.int32)])
        dst = jnp.concatenate([dst, jnp.full((ep - e,), n, jnp.int32)])
    src2 = src * 2
    src0 = src2.reshape(-1, _ROW)
    src1 = (src2 + 1).reshape(-1, _ROW)
    dst = dst.reshape(-1, _ROW)

    eye4 = jnp.eye(4, dtype=jnp.float32)
    params = []
    for w1, b1, w2, b2 in [(W1_0, b1_0, W2_0, b2_0), (W1_1, b1_1, W2_1, b2_1),
                           (W1_2, b1_2, W2_2, b2_2)]:
        params.append((jnp.kron(eye4, w1), jnp.tile(b1, 4),
                       jnp.kron(eye4, w2), jnp.tile(b2, 4)))
    wfd = jnp.kron(eye4, Wf)
    bfd = jnp.tile(bf, 4)

    segsum = _make_segsum(n, ep)
    mlp = _make_mlp(n // 4, 1000, False)
    mlp_final = _make_mlp(n // 4, 1000, True)

    h4 = x.reshape(n // 4, 128)
    for wd1, b1d, wd2, b2d in params[:2]:
        agg = segsum(h4.reshape(2 * n, _LANE), src0, src1, dst)
        h4 = mlp(h4, agg.reshape(n // 4, 128), wd1, b1d, wd2, b2d)
    wd1, b1d, wd2, b2d = params[2]
    agg = segsum(h4.reshape(2 * n, _LANE), src0, src1, dst)
    y4 = mlp_final(h4, agg.reshape(n // 4, 128), wd1, b1d, wd2, b2d, wfd, bfd)
    return y4.reshape(n, 1)


# speedup vs baseline: 1.0540x; 1.0257x over previous
import functools

import jax
import jax.numpy as jnp
from jax import lax
from jax.experimental import pallas as pl
from jax.experimental.pallas import tpu as pltpu
from jax.experimental.pallas import tpu_sc as plsc

_NC = 2
_NS = 16
_LANE = 16
_ROW = 512
_GRP = 1


@functools.lru_cache(maxsize=None)
def _make_segsum(n: int, ep: int):
    rows_per_tile = ep // _ROW // _NS
    groups = rows_per_tile // _GRP
    npt = -(-n // (_NS * 128)) * 128
    zr = 128
    zcopies = npt // zr
    mesh = plsc.VectorSubcoreMesh(core_axis_name="c", subcore_axis_name="s")

    @functools.partial(
        pl.kernel,
        out_type=jax.ShapeDtypeStruct((n, _NC, _LANE), jnp.float32),
        mesh=mesh,
        scratch_types=[
            pltpu.VMEM_SHARED((n + 8, _LANE), jnp.float32),
            pltpu.VMEM((2, _ROW), jnp.int32),
            pltpu.VMEM((3, _ROW), jnp.int32),
            pltpu.VMEM((2, _ROW, _LANE), jnp.float32),
            pltpu.VMEM((zr, _LANE), jnp.float32),
            pltpu.SemaphoreType.DMA((2,)),
            pltpu.SemaphoreType.DMA((3,)),
            pltpu.SemaphoreType.DMA((2,)),
            pltpu.SemaphoreType.DMA((2,)),
        ],
        compiler_params=pltpu.CompilerParams(use_tc_tiling_on_sc=False),
    )
    def segsum(hp2, src0r, src1r, dstr, out, acc, srcb, dstb, rows, zbuf,
               issem, idsem, gsem, ssem):
        c = lax.axis_index("c")
        s = lax.axis_index("s")
        start = pl.multiple_of(jnp.minimum(s * npt, n - npt), 8)

        def zero_body(i, carry):
            zbuf[i] = jnp.zeros((_LANE,), jnp.float32)
            return carry

        lax.fori_loop(0, zr, zero_body, 0)
        for k in range(zcopies):
            pltpu.sync_copy(zbuf, acc.at[pl.ds(start + k * zr, zr)])
        plsc.subcore_barrier()

        def fire_idx(g, p2, p3):
            row0 = s * rows_per_tile + g

            @pl.when(c == 0)
            def _():
                pltpu.async_copy(src0r.at[row0], srcb.at[p2], issem.at[p2])

            @pl.when(c == 1)
            def _():
                pltpu.async_copy(src1r.at[row0], srcb.at[p2], issem.at[p2])

            pltpu.async_copy(dstr.at[row0], dstb.at[p3], idsem.at[p3])

        def wait_idx(p2, p3):
            pltpu.make_async_copy(src0r.at[0], srcb.at[p2], issem.at[p2]).wait()
            pltpu.make_async_copy(dstr.at[0], dstb.at[p3], idsem.at[p3]).wait()

        def fire_gather(p2):
            pltpu.async_copy(hp2.at[srcb.at[p2]], rows.at[p2], gsem.at[p2])

        def drain_gather(p2):
            pltpu.make_async_copy(hp2.at[srcb.at[p2]], rows.at[p2],
                                  gsem.at[p2]).wait()

        def fire_scatter(p2, p3):
            pltpu.async_copy(rows.at[p2], acc.at[dstb.at[p3]], ssem.at[p2],
                             add=True)

        def drain_scatter(p2):
            pltpu.make_async_copy(rows.at[p2], acc.at[pl.ds(0, _ROW)],
                                  ssem.at[p2]).wait()

        fire_idx(0, 0, 0)
        wait_idx(0, 0)
        fire_gather(0)
        fire_idx(1, 1, 1)

        def group_body(g, carry):
            p2 = lax.rem(g, 2)
            q2 = 1 - p2
            p3 = lax.rem(g, 3)
            pm3 = lax.rem(g + 2, 3)
            pn3 = lax.rem(g + 1, 3)

            @pl.when(g >= 2)
            def _():
                drain_scatter(p2)

            drain_gather(q2)
            fire_scatter(q2, pm3)
            @pl.when(g + 1 < groups)
            def _():
                fire_idx(g + 1, q2, pn3)

            wait_idx(p2, p3)
            fire_gather(p2)
            return carry

        lax.fori_loop(1, groups, group_body, 0)
        last2 = (groups - 1) % 2
        last3 = (groups - 1) % 3
        drain_gather(last2)
        fire_scatter(last2, last3)
        drain_scatter(1 - last2)
        drain_scatter(last2)
        plsc.subcore_barrier()
        pltpu.sync_copy(acc.at[pl.ds(start, npt)],
                        out.at[pl.ds(start, npt), c])

    return segsum


def _mlp_block(h_ref, agg_ref, wd1_ref, b1d_ref, wd2_ref, b2d_ref):
    z = h_ref[...] + agg_ref[...]
    t = jnp.dot(z, wd1_ref[...], preferred_element_type=jnp.float32) + b1d_ref[...]
    t = jnp.maximum(t, 0.0)
    u = jnp.dot(t, wd2_ref[...], preferred_element_type=jnp.float32) + b2d_ref[...]
    return jnp.maximum(u, 0.0)


def _mlp_body(h_ref, agg_ref, wd1_ref, b1d_ref, wd2_ref, b2d_ref, out_ref):
    out_ref[...] = _mlp_block(h_ref, agg_ref, wd1_ref, b1d_ref, wd2_ref, b2d_ref)


def _mlp_final_body(h_ref, agg_ref, wd1_ref, b1d_ref, wd2_ref, b2d_ref,
                    wfd_ref, bfd_ref, y_ref):
    u = _mlp_block(h_ref, agg_ref, wd1_ref, b1d_ref, wd2_ref, b2d_ref)
    y_ref[...] = jnp.dot(u, wfd_ref[...], preferred_element_type=jnp.float32) + bfd_ref[...]


@functools.lru_cache(maxsize=None)
def _make_mlp(n4: int, blk: int, final: bool):
    grid = (n4 // blk,)
    h_spec = pl.BlockSpec((blk, 128), lambda i: (i, 0))
    w_spec = pl.BlockSpec((128, 128), lambda i: (0, 0))
    b_spec = pl.BlockSpec((128,), lambda i: (0,))
    in_specs = [h_spec, h_spec, w_spec, b_spec, w_spec, b_spec]
    if final:
        in_specs += [pl.BlockSpec((128, 4), lambda i: (0, 0)),
                     pl.BlockSpec((4,), lambda i: (0,))]
        out_spec = pl.BlockSpec((blk, 4), lambda i: (i, 0))
        out_shape = jax.ShapeDtypeStruct((n4, 4), jnp.float32)
        body = _mlp_final_body
    else:
        out_spec = h_spec
        out_shape = jax.ShapeDtypeStruct((n4, 128), jnp.float32)
        body = _mlp_body
    return pl.pallas_call(
        body, grid=grid, in_specs=in_specs, out_specs=out_spec, out_shape=out_shape
    )


def kernel(x, edge_index, W1_0, b1_0, W2_0, b2_0, W1_1, b1_1, W2_1, b2_1,
           W1_2, b1_2, W2_2, b2_2, Wf, bf):
    n, f = x.shape
    e = edge_index.shape[1]
    assert f == 32 and n % 8 == 0

    tile_quant = _NS * _ROW * _GRP
    ep = -(-e // tile_quant) * tile_quant
    src = edge_index[0]
    dst = edge_index[1]
    if ep != e:
        src = jnp.concatenate([src, jnp.zeros((ep - e,), jnp.int32)])
        dst = jnp.concatenate([dst, jnp.full((ep - e,), n, jnp.int32)])
    src2 = src * 2
    src0 = src2.reshape(-1, _ROW)
    src1 = (src2 + 1).reshape(-1, _ROW)
    dst = dst.reshape(-1, _ROW)

    eye4 = jnp.eye(4, dtype=jnp.float32)
    params = []
    for w1, b1, w2, b2 in [(W1_0, b1_0, W2_0, b2_0), (W1_1, b1_1, W2_1, b2_1),
                           (W1_2, b1_2, W2_2, b2_2)]:
        params.append((jnp.kron(eye4, w1), jnp.tile(b1, 4),
                       jnp.kron(eye4, w2), jnp.tile(b2, 4)))
    wfd = jnp.kron(eye4, Wf)
    bfd = jnp.tile(bf, 4)

    segsum = _make_segsum(n, ep)
    blk = 5000 if (n // 4) % 5000 == 0 else 1000
    mlp = _make_mlp(n // 4, blk, False)
    mlp_final = _make_mlp(n // 4, blk, True)

    h4 = x.reshape(n // 4, 128)
    for wd1, b1d, wd2, b2d in params[:2]:
        agg = segsum(h4.reshape(2 * n, _LANE), src0, src1, dst)
        h4 = mlp(h4, agg.reshape(n // 4, 128), wd1, b1d, wd2, b2d)
    wd1, b1d, wd2, b2d = params[2]
    agg = segsum(h4.reshape(2 * n, _LANE), src0, src1, dst)
    y4 = mlp_final(h4, agg.reshape(n // 4, 128), wd1, b1d, wd2, b2d, wfd, bfd)
    return y4.reshape(n, 1)
